# Initial kernel scaffold; baseline (speedup 1.0000x reference)
import functools

import jax
import jax.numpy as jnp
from jax import lax
from jax.experimental import pallas as pl
from jax.experimental.pallas import tpu as pltpu
from jax.experimental.pallas import tpu_sc as plsc

EMBED_DIM = 256
NUM_EMB = 8192
COMMIT = 0.25

M = 8192
MB = 256
NM = M // MB
PB = 1024 // MB

_SC_CORES = 2
_SC_SUBCORES = 16
_SC_WORKERS = _SC_CORES * _SC_SUBCORES
_BPW = M // _SC_WORKERS


def _argmin_body(x2_ref, e2_ref, x_ref, e_ref, idx_ref):
    mm = jnp.dot(x_ref[...], e_ref[...], preferred_element_type=jnp.float32)
    dist = (x2_ref[...] + e2_ref[...]) - 2.0 * mm
    idx_ref[0, 0, :] = jnp.argmin(dist, axis=1).astype(jnp.int32)


def _onehot_body(idx_ref, enc_ref, avg_ref, cnt_ref):
    p = pl.program_id(0)
    b = pl.program_id(1)
    idx = idx_ref[0, 0, :]
    iota = lax.broadcasted_iota(jnp.int32, (MB, NUM_EMB), 1)
    oh = jnp.where(idx[:, None] == iota, jnp.float32(1.0), jnp.float32(0.0))
    enc_ref[0] = oh
    contrib = oh * jnp.float32(0.125)

    @pl.when(b == 0)
    def _():
        avg_ref[...] = contrib

    @pl.when(b != 0)
    def _():
        avg_ref[...] += contrib

    s = jnp.sum(oh, axis=0, keepdims=True)
    first = (p == 0) & (b == 0)

    @pl.when(first)
    def _():
        cnt_ref[...] = s

    @pl.when(~first)
    def _():
        cnt_ref[...] += s


def _final_body(x_ref, q_ref, cnt_ref, qst_ref, loss_ref, perp_ref, acc_ref):
    i = pl.program_id(0)
    x = x_ref[...]
    q = q_ref[...]
    d = q - x
    qst_ref[...] = x + d
    s = jnp.sum(d * d)

    @pl.when(i == 0)
    def _():
        acc_ref[0] = s

    @pl.when(i != 0)
    def _():
        acc_ref[0] += s

    @pl.when(i == pl.num_programs(0) - 1)
    def _():
        m = acc_ref[0] / jnp.float32(M * EMBED_DIM)
        loss_ref[0, 0] = m + jnp.float32(COMMIT) * m
        pr = cnt_ref[...] * jnp.float32(1.0 / M)
        ent = jnp.sum(pr * jnp.log(pr + jnp.float32(1e-20)))
        perp_ref[0, 0] = jnp.exp(-ent)


@functools.partial(
    pl.kernel,
    mesh=plsc.VectorSubcoreMesh(core_axis_name="c", subcore_axis_name="s"),
    out_type=jax.ShapeDtypeStruct((M, EMBED_DIM), jnp.float32),
    scratch_types=[
        pltpu.VMEM((_BPW,), jnp.int32),
        pltpu.VMEM((_BPW, EMBED_DIM), jnp.float32),
        pltpu.SemaphoreType.DMA,
    ],
)
def _sc_gather(table_hbm, idx_hbm, out_hbm, idx_v, rows_v, sem):
    wid = lax.axis_index("s") * _SC_CORES + lax.axis_index("c")
    base = wid * _BPW
    pltpu.sync_copy(idx_hbm.at[pl.ds(base, _BPW)], idx_v)
    pltpu.async_copy(table_hbm.at[idx_v], rows_v, sem).wait()
    pltpu.sync_copy(rows_v, out_hbm.at[pl.ds(base, _BPW)])


def kernel(x, embeddings):
    e = embeddings
    input_shape = x.shape[:-1]
    xf = x.reshape((-1, EMBED_DIM))
    x2 = jnp.sum(xf * xf, -1, keepdims=True)
    e2 = jnp.sum(e * e, 0, keepdims=True)
    e_t = jnp.swapaxes(e, 0, 1)

    idx3 = pl.pallas_call(
        _argmin_body,
        grid=(NM,),
        in_specs=[
            pl.BlockSpec((MB, 1), lambda i: (i, 0)),
            pl.BlockSpec((1, NUM_EMB), lambda i: (0, 0)),
            pl.BlockSpec((MB, EMBED_DIM), lambda i: (i, 0)),
            pl.BlockSpec((EMBED_DIM, NUM_EMB), lambda i: (0, 0)),
        ],
        out_specs=pl.BlockSpec((1, 1, MB), lambda i: (i, 0, 0)),
        out_shape=jax.ShapeDtypeStruct((NM, 1, MB), jnp.int32),
    )(x2, e2, xf, e)

    enc3, avg_probs, cnt = pl.pallas_call(
        _onehot_body,
        grid=(PB, 8),
        in_specs=[
            pl.BlockSpec((1, 1, MB), lambda p, b: (b * PB + p, 0, 0)),
        ],
        out_specs=[
            pl.BlockSpec((1, MB, NUM_EMB), lambda p, b: (b, p, 0)),
            pl.BlockSpec((MB, NUM_EMB), lambda p, b: (p, 0)),
            pl.BlockSpec((1, NUM_EMB), lambda p, b: (0, 0)),
        ],
        out_shape=[
            jax.ShapeDtypeStruct((8, 1024, NUM_EMB), jnp.float32),
            jax.ShapeDtypeStruct((1024, NUM_EMB), jnp.float32),
            jax.ShapeDtypeStruct((1, NUM_EMB), jnp.float32),
        ],
    )(idx3)

    enc_idx = idx3.reshape((M,))
    quantized = _sc_gather(e_t, enc_idx)

    qst, loss, perp = pl.pallas_call(
        _final_body,
        grid=(8,),
        in_specs=[
            pl.BlockSpec((1024, EMBED_DIM), lambda i: (i, 0)),
            pl.BlockSpec((1024, EMBED_DIM), lambda i: (i, 0)),
            pl.BlockSpec((1, NUM_EMB), lambda i: (0, 0)),
        ],
        out_specs=[
            pl.BlockSpec((1024, EMBED_DIM), lambda i: (i, 0)),
            pl.BlockSpec((1, 1), lambda i: (0, 0)),
            pl.BlockSpec((1, 1), lambda i: (0, 0)),
        ],
        out_shape=[
            jax.ShapeDtypeStruct((M, EMBED_DIM), jnp.float32),
            jax.ShapeDtypeStruct((1, 1), jnp.float32),
            jax.ShapeDtypeStruct((1, 1), jnp.float32),
        ],
        scratch_shapes=[pltpu.SMEM((1,), jnp.float32)],
    )(xf, quantized, cnt)

    aux = {
        "encoding": enc3.reshape((M, NUM_EMB)),
        "encoding_index": enc_idx,
        "avg_probs": avg_probs,
        "perplexity": perp[0, 0],
        "centers": e_t,
    }
    return (qst.reshape((*input_shape, EMBED_DIM)), loss[0, 0], aux)


# speedup vs baseline: 1.2965x; 1.2965x over previous
import functools

import jax
import jax.numpy as jnp
from jax import lax
from jax.experimental import pallas as pl
from jax.experimental.pallas import tpu as pltpu
from jax.experimental.pallas import tpu_sc as plsc

EMBED_DIM = 256
NUM_EMB = 8192
COMMIT = 0.25

M = 8192
MB = 256
NM = M // MB
PB = 1024 // MB

_SC_CORES = 2
_SC_SUBCORES = 16
_SC_WORKERS = _SC_CORES * _SC_SUBCORES
_BPW = M // _SC_WORKERS


def _argmin_body(x2_ref, e2_ref, x_ref, e_ref, idx_ref):
    mm = lax.dot_general(
        x_ref[...], e_ref[...],
        (((1,), (0,)), ((), ())),
        preferred_element_type=jnp.float32,
    )
    dist = (x2_ref[...] + e2_ref[...]) - mm
    idx_ref[0, 0, :] = jnp.argmin(dist, axis=1).astype(jnp.int32)


def _onehot_body(idx_ref, enc_ref, avg_ref, cnt_ref):
    p = pl.program_id(0)
    b = pl.program_id(1)
    idx = idx_ref[0, 0, :]
    iota = lax.broadcasted_iota(jnp.int32, (MB, NUM_EMB), 1)
    oh = jnp.where(idx[:, None] == iota, jnp.float32(1.0), jnp.float32(0.0))
    enc_ref[0] = oh
    contrib = oh * jnp.float32(0.125)

    @pl.when(b == 0)
    def _():
        avg_ref[...] = contrib

    @pl.when(b != 0)
    def _():
        avg_ref[...] += contrib

    s = jnp.sum(oh, axis=0, keepdims=True)
    first = (p == 0) & (b == 0)

    @pl.when(first)
    def _():
        cnt_ref[...] = s

    @pl.when(~first)
    def _():
        cnt_ref[...] += s


def _final_body(x_ref, q_ref, cnt_ref, qst_ref, loss_ref, perp_ref, acc_ref):
    i = pl.program_id(0)
    x = x_ref[...]
    q = q_ref[...]
    d = q - x
    qst_ref[...] = x + d
    s = jnp.sum(d * d)

    @pl.when(i == 0)
    def _():
        acc_ref[0] = s

    @pl.when(i != 0)
    def _():
        acc_ref[0] += s

    @pl.when(i == pl.num_programs(0) - 1)
    def _():
        m = acc_ref[0] / jnp.float32(M * EMBED_DIM)
        loss_ref[...] = jnp.reshape(m + jnp.float32(COMMIT) * m, (1, 1))
        pr = cnt_ref[...] * jnp.float32(1.0 / M)
        ent = jnp.sum(pr * jnp.log(pr + jnp.float32(1e-20)))
        perp_ref[...] = jnp.reshape(jnp.exp(-ent), (1, 1))


@functools.cache
def _get_sc_gather():
    mesh = plsc.VectorSubcoreMesh(
        core_axis_name="c", subcore_axis_name="s",
        num_cores=_SC_CORES, num_subcores=_SC_SUBCORES)

    @functools.partial(
        pl.kernel,
        mesh=mesh,
        out_type=jax.ShapeDtypeStruct((M, EMBED_DIM), jnp.float32),
        scratch_types=[
            pltpu.VMEM((_BPW,), jnp.int32),
            pltpu.VMEM((_BPW, EMBED_DIM), jnp.float32),
            pltpu.SemaphoreType.DMA,
        ],
    )
    def _sc_gather(table_hbm, idx_hbm, out_hbm, idx_v, rows_v, sem):
        wid = lax.axis_index("s") * _SC_CORES + lax.axis_index("c")
        base = wid * _BPW
        pltpu.sync_copy(idx_hbm.at[pl.ds(base, _BPW)], idx_v)
        pltpu.async_copy(table_hbm.at[idx_v], rows_v, sem).wait()
        pltpu.sync_copy(rows_v, out_hbm.at[pl.ds(base, _BPW)])

    return _sc_gather


def kernel(x, embeddings):
    e = embeddings
    input_shape = x.shape[:-1]
    xf = x.reshape((-1, EMBED_DIM))
    x2 = jnp.sum(xf * xf, -1, keepdims=True)
    e2 = jnp.sum(e * e, 0, keepdims=True)
    x2bf = (2.0 * xf).astype(jnp.bfloat16)
    e_t = jnp.swapaxes(e, 0, 1)

    idx3 = pl.pallas_call(
        _argmin_body,
        grid=(NM,),
        in_specs=[
            pl.BlockSpec((MB, 1), lambda i: (i, 0)),
            pl.BlockSpec((1, NUM_EMB), lambda i: (0, 0)),
            pl.BlockSpec((MB, EMBED_DIM), lambda i: (i, 0)),
            pl.BlockSpec((EMBED_DIM, NUM_EMB), lambda i: (0, 0)),
        ],
        out_specs=pl.BlockSpec((1, 1, MB), lambda i: (i, 0, 0)),
        out_shape=jax.ShapeDtypeStruct((NM, 1, MB), jnp.int32),
    )(x2, e2, x2bf, e)

    enc3, avg_probs, cnt = pl.pallas_call(
        _onehot_body,
        grid=(PB, 8),
        in_specs=[
            pl.BlockSpec((1, 1, MB), lambda p, b: (b * PB + p, 0, 0)),
        ],
        out_specs=[
            pl.BlockSpec((1, MB, NUM_EMB), lambda p, b: (b, p, 0)),
            pl.BlockSpec((MB, NUM_EMB), lambda p, b: (p, 0)),
            pl.BlockSpec((1, NUM_EMB), lambda p, b: (0, 0)),
        ],
        out_shape=[
            jax.ShapeDtypeStruct((8, 1024, NUM_EMB), jnp.float32),
            jax.ShapeDtypeStruct((1024, NUM_EMB), jnp.float32),
            jax.ShapeDtypeStruct((1, NUM_EMB), jnp.float32),
        ],
    )(idx3)

    enc_idx = idx3.reshape((M,))
    quantized = _get_sc_gather()(e_t, enc_idx)

    qst, loss, perp = pl.pallas_call(
        _final_body,
        grid=(8,),
        in_specs=[
            pl.BlockSpec((1024, EMBED_DIM), lambda i: (i, 0)),
            pl.BlockSpec((1024, EMBED_DIM), lambda i: (i, 0)),
            pl.BlockSpec((1, NUM_EMB), lambda i: (0, 0)),
        ],
        out_specs=[
            pl.BlockSpec((1024, EMBED_DIM), lambda i: (i, 0)),
            pl.BlockSpec((1, 1), lambda i: (0, 0)),
            pl.BlockSpec((1, 1), lambda i: (0, 0)),
        ],
        out_shape=[
            jax.ShapeDtypeStruct((M, EMBED_DIM), jnp.float32),
            jax.ShapeDtypeStruct((1, 1), jnp.float32),
            jax.ShapeDtypeStruct((1, 1), jnp.float32),
        ],
        scratch_shapes=[pltpu.SMEM((1,), jnp.float32)],
    )(xf, quantized, cnt)

    aux = {
        "encoding": enc3.reshape((M, NUM_EMB)),
        "encoding_index": enc_idx,
        "avg_probs": avg_probs,
        "perplexity": perp[0, 0],
        "centers": e_t,
    }
    return (qst.reshape((*input_shape, EMBED_DIM)), loss[0, 0], aux)
